# baseline (speedup 1.0000x reference)
import jax
import jax.numpy as jnp
from jax import lax
from jax.experimental import pallas as pl
from jax.experimental.pallas import tpu as pltpu

_R4 = 1000
_R = 4 * _R4
_N = 100000
_NCHUNK = _N // _R
_DEPTH = 4
_NSLOT = _DEPTH + 1
_OSLOT = 4


def _in_copy(x_hbm, x_buf, isem, k, slot, j):
    return pltpu.make_async_copy(
        x_hbm.at[pl.ds(k * _R + j * _R4, _R4), :],
        x_buf.at[slot, pl.ds(j * _R4, _R4), :],
        isem.at[slot, j],
    )


def _out_copy(o_hbm, o_buf, osem, k, oslot):
    return pltpu.make_async_copy(
        o_buf.at[oslot],
        o_hbm.at[pl.ds(k * _R, _R), :],
        osem.at[oslot],
    )


def _mlp_body(x_hbm, w1_ref, b1_ref, w2c_ref, b2c_ref, o_hbm,
              x_buf, o_buf, isem, osem):
    c = pl.program_id(0)

    @pl.when(c == 0)
    def _():
        for k in range(_DEPTH):
            for j in range(4):
                _in_copy(x_hbm, x_buf, isem, k, k, j).start()

    @pl.when(c + _DEPTH < _NCHUNK)
    def _():
        k = c + _DEPTH
        slot = lax.rem(k, _NSLOT)
        for j in range(4):
            _in_copy(x_hbm, x_buf, isem, k, slot, j).start()

    slot = lax.rem(c, _NSLOT)
    for j in range(4):
        _in_copy(x_hbm, x_buf, isem, c, slot, j).wait()

    x = x_buf[slot]
    h1t = lax.dot_general(w1_ref[...], x, (((1,), (1,)), ((), ())),
                          preferred_element_type=jnp.float32)
    h1t = jnp.maximum(h1t + b1_ref[...], 0.0)
    lt = lax.dot_general(w2c_ref[...], h1t, (((1,), (0,)), ((), ())),
                         preferred_element_type=jnp.float32)
    lt = lt + b2c_ref[...]
    lse = jnp.log(jnp.sum(jnp.exp(lt), axis=0, keepdims=True))
    res = (lt - lse).T

    oslot = lax.rem(c, _OSLOT)

    @pl.when(c >= _OSLOT)
    def _():
        _out_copy(o_hbm, o_buf, osem, c - _OSLOT, oslot).wait()

    o_buf[oslot] = res
    _out_copy(o_hbm, o_buf, osem, c, oslot).start()

    @pl.when(c == _NCHUNK - 1)
    def _():
        for k in range(_NCHUNK - _OSLOT, _NCHUNK):
            _out_copy(o_hbm, o_buf, osem, k, k % _OSLOT).wait()


@jax.jit
def _run(x, w1, b1, w2c, b2c):
    return pl.pallas_call(
        _mlp_body,
        grid=(_NCHUNK,),
        in_specs=[
            pl.BlockSpec(memory_space=pl.ANY),
            pl.BlockSpec((64, 128), lambda i: (0, 0)),
            pl.BlockSpec((64, 1), lambda i: (0, 0)),
            pl.BlockSpec((32, 64), lambda i: (0, 0)),
            pl.BlockSpec((32, 1), lambda i: (0, 0)),
        ],
        out_specs=pl.BlockSpec(memory_space=pl.ANY),
        out_shape=jax.ShapeDtypeStruct((_N, 32), jnp.float32),
        scratch_shapes=[
            pltpu.VMEM((_NSLOT, _R, 128), jnp.float32),
            pltpu.VMEM((_OSLOT, _R, 32), jnp.float32),
            pltpu.SemaphoreType.DMA((_NSLOT, 4)),
            pltpu.SemaphoreType.DMA((_OSLOT,)),
        ],
        compiler_params=pltpu.CompilerParams(
            dimension_semantics=("arbitrary",)),
    )(x, w1, b1, w2c, b2c)


def kernel(x, W1, b1, W2, b2):
    cen = (jnp.eye(32, dtype=jnp.float32)
           - jnp.full((32, 32), 1.0 / 32.0, jnp.float32))
    w2c = cen @ W2
    b2c = (cen @ b2).reshape(-1, 1)
    return _run(x, W1, b1.reshape(-1, 1), w2c, b2c)


# speedup vs baseline: 1.0245x; 1.0245x over previous
import jax
import jax.numpy as jnp
from jax import lax
from jax.experimental import pallas as pl
from jax.experimental.pallas import tpu as pltpu

_R4 = 1000
_R = 4 * _R4
_N = 100000
_NCHUNK = _N // _R
_DEPTH = 6
_NSLOT = _DEPTH + 1


def _in_copy(x_hbm, x_buf, isem, k, slot, j):
    return pltpu.make_async_copy(
        x_hbm.at[pl.ds(k * _R + j * _R4, _R4), :],
        x_buf.at[slot, pl.ds(j * _R4, _R4), :],
        isem.at[slot, j],
    )


def _mlp_body(x_hbm, w1_ref, b1_ref, w2c_ref, b2c_ref, o_ref,
              x_buf, isem):
    c = pl.program_id(0)

    @pl.when(c == 0)
    def _():
        for k in range(_DEPTH):
            for j in range(4):
                _in_copy(x_hbm, x_buf, isem, k, k, j).start()

    @pl.when(c + _DEPTH < _NCHUNK)
    def _():
        k = c + _DEPTH
        slot = lax.rem(k, _NSLOT)
        for j in range(4):
            _in_copy(x_hbm, x_buf, isem, k, slot, j).start()

    slot = lax.rem(c, _NSLOT)
    for j in range(4):
        _in_copy(x_hbm, x_buf, isem, c, slot, j).wait()

    x = x_buf[slot]
    h1t = lax.dot_general(w1_ref[...], x, (((1,), (1,)), ((), ())),
                          preferred_element_type=jnp.float32)
    h1t = jnp.maximum(h1t + b1_ref[...], 0.0)
    lt = lax.dot_general(w2c_ref[...], h1t, (((1,), (0,)), ((), ())),
                         preferred_element_type=jnp.float32)
    lt = lt + b2c_ref[...]
    lse = jnp.log(jnp.sum(jnp.exp(lt), axis=0, keepdims=True))
    o_ref[...] = (lt - lse).T


@jax.jit
def _run(x, w1, b1, w2c, b2c):
    return pl.pallas_call(
        _mlp_body,
        grid=(_NCHUNK,),
        in_specs=[
            pl.BlockSpec(memory_space=pl.ANY),
            pl.BlockSpec((64, 128), lambda i: (0, 0)),
            pl.BlockSpec((64, 1), lambda i: (0, 0)),
            pl.BlockSpec((32, 64), lambda i: (0, 0)),
            pl.BlockSpec((32, 1), lambda i: (0, 0)),
        ],
        out_specs=pl.BlockSpec((_R, 32), lambda i: (i, 0)),
        out_shape=jax.ShapeDtypeStruct((_N, 32), jnp.float32),
        scratch_shapes=[
            pltpu.VMEM((_NSLOT, _R, 128), jnp.float32),
            pltpu.SemaphoreType.DMA((_NSLOT, 4)),
        ],
        compiler_params=pltpu.CompilerParams(
            dimension_semantics=("arbitrary",)),
    )(x, w1, b1, w2c, b2c)


def kernel(x, W1, b1, W2, b2):
    cen = (jnp.eye(32, dtype=jnp.float32)
           - jnp.full((32, 32), 1.0 / 32.0, jnp.float32))
    w2c = cen @ W2
    b2c = (cen @ b2).reshape(-1, 1)
    return _run(x, W1, b1.reshape(-1, 1), w2c, b2c)


# speedup vs baseline: 1.0411x; 1.0162x over previous
import jax
import jax.numpy as jnp
from jax import lax
from jax.experimental import pallas as pl
from jax.experimental.pallas import tpu as pltpu

_R4 = 5000
_R = 4 * _R4
_N = 100000
_NCHUNK = _N // _R
_DEPTH = 2
_NSLOT = _DEPTH + 1


def _in_copy(x_hbm, x_buf, isem, k, slot, j):
    return pltpu.make_async_copy(
        x_hbm.at[pl.ds(k * _R + j * _R4, _R4), :],
        x_buf.at[slot, pl.ds(j * _R4, _R4), :],
        isem.at[slot, j],
    )


def _mlp_body(x_hbm, w1_ref, b1_ref, w2c_ref, b2c_ref, o_ref,
              x_buf, isem):
    c = pl.program_id(0)

    @pl.when(c == 0)
    def _():
        for k in range(_DEPTH):
            for j in range(4):
                _in_copy(x_hbm, x_buf, isem, k, k, j).start()

    @pl.when(c + _DEPTH < _NCHUNK)
    def _():
        k = c + _DEPTH
        slot = lax.rem(k, _NSLOT)
        for j in range(4):
            _in_copy(x_hbm, x_buf, isem, k, slot, j).start()

    slot = lax.rem(c, _NSLOT)
    for j in range(4):
        _in_copy(x_hbm, x_buf, isem, c, slot, j).wait()

    x = x_buf[slot]
    h1t = lax.dot_general(w1_ref[...], x, (((1,), (1,)), ((), ())),
                          preferred_element_type=jnp.float32)
    h1t = jnp.maximum(h1t + b1_ref[...], 0.0)
    lt = lax.dot_general(w2c_ref[...], h1t, (((1,), (0,)), ((), ())),
                         preferred_element_type=jnp.float32)
    lt = lt + b2c_ref[...]
    lse = jnp.log(jnp.sum(jnp.exp(lt), axis=0, keepdims=True))
    o_ref[...] = (lt - lse).T


@jax.jit
def _run(x, w1, b1, w2c, b2c):
    return pl.pallas_call(
        _mlp_body,
        grid=(_NCHUNK,),
        in_specs=[
            pl.BlockSpec(memory_space=pl.ANY),
            pl.BlockSpec((64, 128), lambda i: (0, 0)),
            pl.BlockSpec((64, 1), lambda i: (0, 0)),
            pl.BlockSpec((32, 64), lambda i: (0, 0)),
            pl.BlockSpec((32, 1), lambda i: (0, 0)),
        ],
        out_specs=pl.BlockSpec((_R, 32), lambda i: (i, 0)),
        out_shape=jax.ShapeDtypeStruct((_N, 32), jnp.float32),
        scratch_shapes=[
            pltpu.VMEM((_NSLOT, _R, 128), jnp.float32),
            pltpu.SemaphoreType.DMA((_NSLOT, 4)),
        ],
        compiler_params=pltpu.CompilerParams(
            dimension_semantics=("arbitrary",)),
    )(x, w1, b1, w2c, b2c)


def kernel(x, W1, b1, W2, b2):
    cen = (jnp.eye(32, dtype=jnp.float32)
           - jnp.full((32, 32), 1.0 / 32.0, jnp.float32))
    w2c = cen @ W2
    b2c = (cen @ b2).reshape(-1, 1)
    return _run(x, W1, b1.reshape(-1, 1), w2c, b2c)


# speedup vs baseline: 1.0442x; 1.0030x over previous
import jax
import jax.numpy as jnp
from jax import lax
from jax.experimental import pallas as pl
from jax.experimental.pallas import tpu as pltpu

_R4 = 5000
_R = 4 * _R4
_N = 100000
_NCHUNK = _N // _R
_DEPTH = 2
_NSLOT = _DEPTH + 1
_OSLOT = 2


def _in_copy(x_hbm, x_buf, isem, k, slot, j):
    return pltpu.make_async_copy(
        x_hbm.at[pl.ds(k * _R + j * _R4, _R4), :],
        x_buf.at[slot, pl.ds(j * _R4, _R4), :],
        isem.at[slot, j],
    )


def _out_copy(o_hbm, o_buf, osem, k, oslot):
    return pltpu.make_async_copy(
        o_buf.at[oslot],
        o_hbm.at[pl.ds(k * _R, _R), :],
        osem.at[oslot],
    )


def _mlp_body(x_hbm, w1_ref, b1_ref, w2c_ref, b2c_ref, o_hbm,
              x_buf, o_buf, isem, osem):
    for k in range(_DEPTH):
        for j in range(4):
            _in_copy(x_hbm, x_buf, isem, k, k % _NSLOT, j).start()
    for c in range(_NCHUNK):
        if c + _DEPTH < _NCHUNK:
            k = c + _DEPTH
            for j in range(4):
                _in_copy(x_hbm, x_buf, isem, k, k % _NSLOT, j).start()
        for j in range(4):
            _in_copy(x_hbm, x_buf, isem, c, c % _NSLOT, j).wait()
        x = x_buf[c % _NSLOT]
        h1t = lax.dot_general(w1_ref[...], x, (((1,), (1,)), ((), ())),
                              preferred_element_type=jnp.float32)
        h1t = jnp.maximum(h1t + b1_ref[...], 0.0)
        lt = lax.dot_general(w2c_ref[...], h1t, (((1,), (0,)), ((), ())),
                             preferred_element_type=jnp.float32)
        lt = lt + b2c_ref[...]
        lse = jnp.log(jnp.sum(jnp.exp(lt), axis=0, keepdims=True))
        res = (lt - lse).T
        if c >= _OSLOT:
            _out_copy(o_hbm, o_buf, osem, c - _OSLOT, c % _OSLOT).wait()
        o_buf[c % _OSLOT] = res
        _out_copy(o_hbm, o_buf, osem, c, c % _OSLOT).start()
    for k in range(max(0, _NCHUNK - _OSLOT), _NCHUNK):
        _out_copy(o_hbm, o_buf, osem, k, k % _OSLOT).wait()

@jax.jit
def _run(x, w1, b1, w2c, b2c):
    return pl.pallas_call(
        _mlp_body,
        grid=(1,),
        in_specs=[
            pl.BlockSpec(memory_space=pl.ANY),
            pl.BlockSpec((64, 128), lambda i: (0, 0)),
            pl.BlockSpec((64, 1), lambda i: (0, 0)),
            pl.BlockSpec((32, 64), lambda i: (0, 0)),
            pl.BlockSpec((32, 1), lambda i: (0, 0)),
        ],
        out_specs=pl.BlockSpec(memory_space=pl.ANY),
        out_shape=jax.ShapeDtypeStruct((_N, 32), jnp.float32),
        scratch_shapes=[
            pltpu.VMEM((_NSLOT, _R, 128), jnp.float32),
            pltpu.VMEM((_OSLOT, _R, 32), jnp.float32),
            pltpu.SemaphoreType.DMA((_NSLOT, 4)),
            pltpu.SemaphoreType.DMA((_OSLOT,)),
        ],
        compiler_params=pltpu.CompilerParams(
            dimension_semantics=("arbitrary",)),
    )(x, w1, b1, w2c, b2c)


def kernel(x, W1, b1, W2, b2):
    cen = (jnp.eye(32, dtype=jnp.float32)
           - jnp.full((32, 32), 1.0 / 32.0, jnp.float32))
    w2c = cen @ W2
    b2c = (cen @ b2).reshape(-1, 1)
    return _run(x, W1, b1.reshape(-1, 1), w2c, b2c)
